# baseline (speedup 1.0000x reference)
import functools

import jax
import jax.numpy as jnp
from jax import lax
from jax.experimental import pallas as pl
from jax.experimental.pallas import tpu as pltpu
from jax.experimental.pallas import tpu_sc as plsc

H = 128
N = 10000
NP = 10240
E = 320000
N2 = 2000
N2P = 2048
NS = 400
G = 16
C = 10

_NWORK = 32
_CHUNK = 80

_mesh = plsc.VectorSubcoreMesh(core_axis_name="c", subcore_axis_name="s")


def _make_sc_gather(n_idx):
    per_w = n_idx // _NWORK
    n_chunks = per_w // _CHUNK

    @functools.partial(
        pl.kernel,
        out_type=jax.ShapeDtypeStruct((n_idx, H), jnp.float32),
        mesh=_mesh,
        scratch_types=[
            pltpu.VMEM((_CHUNK,), jnp.int32),
            pltpu.VMEM((_CHUNK, H), jnp.float32),
            pltpu.SemaphoreType.DMA,
        ],
    )
    def k(table_hbm, idx_hbm, out_hbm, idx_v, rows_v, sem):
        cid = lax.axis_index("c")
        sid = lax.axis_index("s")
        base = (sid * 2 + cid) * per_w

        def body(i, carry):
            off = base + i * _CHUNK
            pltpu.sync_copy(idx_hbm.at[pl.ds(off, _CHUNK)], idx_v)
            pltpu.async_copy(table_hbm.at[idx_v], rows_v, sem).wait()
            pltpu.sync_copy(rows_v, out_hbm.at[pl.ds(off, _CHUNK), :])
            return carry

        lax.fori_loop(0, n_chunks, body, 0)

    return k


def _make_sc_gather_pipe(n_idx):
    per_w = n_idx // _NWORK

    @functools.partial(
        pl.kernel,
        out_type=jax.ShapeDtypeStruct((n_idx, H), jnp.float32),
        mesh=_mesh,
        scratch_types=[
            pltpu.VMEM((_CHUNK,), jnp.int32),
            pltpu.VMEM((_CHUNK,), jnp.int32),
            pltpu.VMEM((_CHUNK, H), jnp.float32),
            pltpu.VMEM((_CHUNK, H), jnp.float32),
            pltpu.SemaphoreType.DMA,
            pltpu.SemaphoreType.DMA,
            pltpu.SemaphoreType.DMA,
            pltpu.SemaphoreType.DMA,
            pltpu.SemaphoreType.DMA,
            pltpu.SemaphoreType.DMA,
        ],
    )
    def k(table_hbm, idx_hbm, out_hbm, ix0, ix1, rw0, rw1,
          i0, i1, g0, g1, o0, o1):
        cid = lax.axis_index("c")
        sid = lax.axis_index("s")
        base = (sid * 2 + cid) * per_w
        ix = [ix0, ix1]
        rw = [rw0, rw1]
        isem = [i0, i1]
        gsem = [g0, g1]
        osem = [o0, o1]

        def off(j):
            return base + j * _CHUNK

        def fi(j):
            pltpu.async_copy(idx_hbm.at[pl.ds(off(j), _CHUNK)], ix[j % 2],
                             isem[j % 2])

        def wi(j):
            pltpu.make_async_copy(idx_hbm.at[pl.ds(off(j), _CHUNK)],
                                  ix[j % 2], isem[j % 2]).wait()

        def fg(j):
            pltpu.async_copy(table_hbm.at[ix[j % 2]], rw[j % 2], gsem[j % 2])

        def wg(j):
            pltpu.make_async_copy(table_hbm.at[ix[j % 2]], rw[j % 2],
                                  gsem[j % 2]).wait()

        def fo(j):
            pltpu.async_copy(rw[j % 2], out_hbm.at[pl.ds(off(j), _CHUNK), :],
                             osem[j % 2])

        def wo(j):
            pltpu.make_async_copy(rw[j % 2],
                                  out_hbm.at[pl.ds(off(j), _CHUNK), :],
                                  osem[j % 2]).wait()

        fi(0); fi(1)
        wi(0); fg(0)
        wi(1); fg(1)
        wg(0); fo(0); fi(2)
        wg(1); fo(1); fi(3)
        wo(0); wi(2); fg(2)
        wo(1); wi(3); fg(3)
        wg(2); fo(2)
        wg(3); fo(3)
        wo(2); wo(3)

    return k


_EC = 80
_EPW = 10240
EP = _NWORK * _EPW
_NCH = _EPW // _EC
_HCH = _NCH // 2
_HPAIR = _HCH // 2


def _make_sc_edge_agg():
    rpt = NP // 16

    @functools.partial(
        pl.kernel,
        out_type=jax.ShapeDtypeStruct((2, NP, H), jnp.float32),
        mesh=_mesh,
        scratch_types=(
            [pltpu.VMEM((_EC,), jnp.int32) for _ in range(16)]
            + [pltpu.VMEM((_EC, H), jnp.float32) for _ in range(4)]
            + [pltpu.VMEM_SHARED((NP, H), jnp.float32)]
            + [pltpu.SemaphoreType.DMA for _ in range(16)]
        ),
    )
    def k(m_hbm, src_hbm, dst_hbm, zero_hbm, out_hbm,
          si0, si1, si2, si3, si4, si5, si6, si7,
          di0, di1, di2, di3, di4, di5, di6, di7,
          rw0, rw1, rw2, rw3, acc,
          is0, is1, is2, is3, is4, is5, is6, is7,
          gg0, gg1, gg2, gg3, ss0, ss1, ss2, ss3):
        cid = lax.axis_index("c")
        sid = lax.axis_index("s")
        wid = sid * 2 + cid
        r0 = sid * rpt
        srcb = [si0, si1, si2, si3, si4, si5, si6, si7]
        dstb = [di0, di1, di2, di3, di4, di5, di6, di7]
        rows = [rw0, rw1, rw2, rw3]
        isem = [is0, is1, is2, is3, is4, is5, is6, is7]
        gsem = [gg0, gg1, gg2, gg3]
        ssem = [ss0, ss1, ss2, ss3]

        def fi(c, p):
            pltpu.async_copy(src_hbm.at[wid, c], srcb[p], isem[p])
            pltpu.async_copy(dst_hbm.at[wid, c], dstb[p], isem[p])

        def wi(c, p):
            pltpu.make_async_copy(src_hbm.at[wid, c], srcb[p],
                                  isem[p]).wait()
            pltpu.make_async_copy(dst_hbm.at[wid, c], dstb[p],
                                  isem[p]).wait()

        def fg(b, p):
            pltpu.async_copy(m_hbm.at[srcb[p]], rows[b], gsem[b])

        def wg(b, p):
            pltpu.make_async_copy(m_hbm.at[srcb[p]], rows[b], gsem[b]).wait()

        def fs(b, p):
            pltpu.async_copy(rows[b], acc.at[dstb[p]], ssem[b], add=True)

        def ws(b, p):
            pltpu.make_async_copy(rows[b], acc.at[dstb[p]], ssem[b]).wait()

        pltpu.sync_copy(zero_hbm.at[pl.ds(r0, rpt), :],
                        acc.at[pl.ds(r0, rpt), :])
        for p in range(6):
            fi(p, p)
        wi(0, 0)
        fg(0, 0)
        wi(1, 1)
        fg(1, 1)
        plsc.subcore_barrier()

        wg(0, 0); fs(0, 0); fi(6, 6); wi(2, 2); fg(2, 2)
        wg(1, 1); fs(1, 1); fi(7, 7); wi(3, 3); fg(3, 3)

        def body(i, carry):
            cbase = 2 + 8 * i
            for j in range(8):
                c = cbase + j
                b = (2 + j) % 4
                q = (2 + j) % 8
                bn = j % 4
                qn = (4 + j) % 8
                wg(b, q)
                fs(b, q)
                ws(bn, j)
                fi(c + 6, j)
                wi(c + 2, qn)
                fg(bn, qn)
            return carry

        lax.fori_loop(0, (_NCH - 8) // 8, body, 0)

        wg(2, 2); fs(2, 2); ws(0, 0); wi(124, 4); fg(0, 4)
        wg(3, 3); fs(3, 3); ws(1, 1); wi(125, 5); fg(1, 5)
        wg(0, 4); fs(0, 4); ws(2, 2); wi(126, 6); fg(2, 6)
        wg(1, 5); fs(1, 5); ws(3, 3); wi(127, 7); fg(3, 7)
        wg(2, 6); fs(2, 6); ws(0, 4)
        wg(3, 7); fs(3, 7); ws(1, 5)
        ws(2, 6)
        ws(3, 7)

        plsc.subcore_barrier()
        pltpu.sync_copy(acc.at[pl.ds(r0, rpt), :],
                        out_hbm.at[cid, pl.ds(r0, rpt), :])

    return k


def _make_sc_segsum(n_rows, n_seg):
    per_w = n_rows // _NWORK
    n_chunks = per_w // _CHUNK
    rpt = n_seg // 16

    @functools.partial(
        pl.kernel,
        out_type=jax.ShapeDtypeStruct((2, n_seg, H), jnp.float32),
        mesh=_mesh,
        scratch_types=[
            pltpu.VMEM((_CHUNK,), jnp.int32),
            pltpu.VMEM((_CHUNK,), jnp.int32),
            pltpu.VMEM((_CHUNK, H), jnp.float32),
            pltpu.VMEM((_CHUNK, H), jnp.float32),
            pltpu.VMEM_SHARED((n_seg, H), jnp.float32),
            pltpu.SemaphoreType.DMA,
            pltpu.SemaphoreType.DMA,
            pltpu.SemaphoreType.DMA,
            pltpu.SemaphoreType.DMA,
        ],
    )
    def k(x_hbm, idx_hbm, zero_hbm, out_hbm, ix0, ix1, rw0, rw1, acc,
          i0, i1, s0, s1):
        cid = lax.axis_index("c")
        sid = lax.axis_index("s")
        r0 = sid * rpt
        base = (sid * 2 + cid) * per_w
        ix = [ix0, ix1]
        rw = [rw0, rw1]
        isem = [i0, i1]
        ssem = [s0, s1]

        def off(j):
            return base + j * _CHUNK

        def fl(j):
            pltpu.async_copy(idx_hbm.at[pl.ds(off(j), _CHUNK)], ix[j % 2],
                             isem[j % 2])
            pltpu.async_copy(x_hbm.at[pl.ds(off(j), _CHUNK), :], rw[j % 2],
                             isem[j % 2])

        def wl(j):
            pltpu.make_async_copy(idx_hbm.at[pl.ds(off(j), _CHUNK)],
                                  ix[j % 2], isem[j % 2]).wait()
            pltpu.make_async_copy(x_hbm.at[pl.ds(off(j), _CHUNK), :],
                                  rw[j % 2], isem[j % 2]).wait()

        def fs(j):
            pltpu.async_copy(rw[j % 2], acc.at[ix[j % 2]], ssem[j % 2],
                             add=True)

        def ws(j):
            pltpu.make_async_copy(rw[j % 2], acc.at[ix[j % 2]],
                                  ssem[j % 2]).wait()

        fl(0); fl(1)
        pltpu.sync_copy(zero_hbm.at[pl.ds(r0, rpt), :],
                        acc.at[pl.ds(r0, rpt), :])
        plsc.subcore_barrier()
        wl(0); fs(0)
        wl(1); fs(1)
        ws(0); fl(2)
        ws(1); fl(3)
        wl(2); fs(2)
        wl(3); fs(3)
        ws(2); ws(3)
        plsc.subcore_barrier()
        pltpu.sync_copy(acc.at[pl.ds(r0, rpt), :],
                        out_hbm.at[cid, pl.ds(r0, rpt), :])

    return k


_sc_emb_gather = _make_sc_gather_pipe(NP)
_sc_edge_agg = _make_sc_edge_agg()
_sc_segsum = _make_sc_segsum(NP, N2P)


_BN = 1024


def _tc_relu_mm(zraw, w):
    def body(z_ref, w_ref, zf_ref, m_ref):
        zf = jnp.maximum(z_ref[...], 0.0)
        zf_ref[...] = zf
        m_ref[...] = jnp.dot(zf, w_ref[...], preferred_element_type=jnp.float32)

    return pl.pallas_call(
        body,
        grid=(NP // _BN,),
        in_specs=[pl.BlockSpec((_BN, H), lambda i: (i, 0)),
                  pl.BlockSpec((H, H), lambda i: (0, 0))],
        out_specs=[pl.BlockSpec((_BN, H), lambda i: (i, 0)),
                   pl.BlockSpec((_BN, H), lambda i: (i, 0))],
        out_shape=[jax.ShapeDtypeStruct((NP, H), jnp.float32),
                   jax.ShapeDtypeStruct((NP, H), jnp.float32)],
    )(zraw, w)


def _tc_mm(zf, w):
    def body(z_ref, w_ref, m_ref):
        m_ref[...] = jnp.dot(z_ref[...], w_ref[...],
                             preferred_element_type=jnp.float32)

    return pl.pallas_call(
        body,
        grid=(NP // _BN,),
        in_specs=[pl.BlockSpec((_BN, H), lambda i: (i, 0)),
                  pl.BlockSpec((H, H), lambda i: (0, 0))],
        out_specs=pl.BlockSpec((_BN, H), lambda i: (i, 0)),
        out_shape=jax.ShapeDtypeStruct((NP, H), jnp.float32),
    )(zf, w)


def _tc_gru(aggp, h, wihT, whhT, bih, bhh):
    def body(p_ref, h_ref, wih_ref, whh_ref, bih_ref, bhh_ref, o_ref):
        agg = p_ref[0] + p_ref[1]
        hh = h_ref[...]
        gi = jnp.dot(agg, wih_ref[...],
                     preferred_element_type=jnp.float32) + bih_ref[...]
        gh = jnp.dot(hh, whh_ref[...],
                     preferred_element_type=jnp.float32) + bhh_ref[...]
        r = jax.nn.sigmoid(gi[:, :H] + gh[:, :H])
        u = jax.nn.sigmoid(gi[:, H:2 * H] + gh[:, H:2 * H])
        nn_ = jnp.tanh(gi[:, 2 * H:] + r * gh[:, 2 * H:])
        o_ref[...] = jnp.maximum((1.0 - u) * nn_ + u * hh, 0.0)

    return pl.pallas_call(
        body,
        grid=(NP // _BN,),
        in_specs=[pl.BlockSpec((2, _BN, H), lambda i: (0, i, 0)),
                  pl.BlockSpec((_BN, H), lambda i: (i, 0)),
                  pl.BlockSpec((H, 3 * H), lambda i: (0, 0)),
                  pl.BlockSpec((H, 3 * H), lambda i: (0, 0)),
                  pl.BlockSpec((1, 3 * H), lambda i: (0, 0)),
                  pl.BlockSpec((1, 3 * H), lambda i: (0, 0))],
        out_specs=pl.BlockSpec((_BN, H), lambda i: (i, 0)),
        out_shape=jax.ShapeDtypeStruct((NP, H), jnp.float32),
    )(aggp, h, wihT, whhT, bih, bhh)


def _tc_gru_mm(aggp, h, wihT, whhT, bih, bhh, wnext):
    def body(p_ref, h_ref, wih_ref, whh_ref, bih_ref, bhh_ref, wn_ref,
             o_ref, m_ref):
        agg = p_ref[0] + p_ref[1]
        hh = h_ref[...]
        gi = jnp.dot(agg, wih_ref[...],
                     preferred_element_type=jnp.float32) + bih_ref[...]
        gh = jnp.dot(hh, whh_ref[...],
                     preferred_element_type=jnp.float32) + bhh_ref[...]
        r = jax.nn.sigmoid(gi[:, :H] + gh[:, :H])
        u = jax.nn.sigmoid(gi[:, H:2 * H] + gh[:, H:2 * H])
        nn_ = jnp.tanh(gi[:, 2 * H:] + r * gh[:, 2 * H:])
        zf = jnp.maximum((1.0 - u) * nn_ + u * hh, 0.0)
        o_ref[...] = zf
        m_ref[...] = jnp.dot(zf, wn_ref[...],
                             preferred_element_type=jnp.float32)

    return pl.pallas_call(
        body,
        grid=(NP // _BN,),
        in_specs=[pl.BlockSpec((2, _BN, H), lambda i: (0, i, 0)),
                  pl.BlockSpec((_BN, H), lambda i: (i, 0)),
                  pl.BlockSpec((H, 3 * H), lambda i: (0, 0)),
                  pl.BlockSpec((H, 3 * H), lambda i: (0, 0)),
                  pl.BlockSpec((1, 3 * H), lambda i: (0, 0)),
                  pl.BlockSpec((1, 3 * H), lambda i: (0, 0)),
                  pl.BlockSpec((H, H), lambda i: (0, 0))],
        out_specs=[pl.BlockSpec((_BN, H), lambda i: (i, 0)),
                   pl.BlockSpec((_BN, H), lambda i: (i, 0))],
        out_shape=[jax.ShapeDtypeStruct((NP, H), jnp.float32),
                   jax.ShapeDtypeStruct((NP, H), jnp.float32)],
    )(aggp, h, wihT, whhT, bih, bhh, wnext)


def _tc_hier(nsp, s2s_pad, s2g, x, pxW, pxb, ew1, eb1, ew2, eb2,
             nw1, nb1, nw2, nb2):
    def body(p_ref, s2s_ref, s2g_ref, x_ref, pxw_ref, pxb_ref,
             ew1_ref, eb1_ref, ew2_ref, eb2_ref,
             nw1_ref, nb1_ref, nw2_ref, nb2_ref, o_ref):
        ne = p_ref[0] + p_ref[1]
        h1 = jnp.maximum(
            jnp.dot(ne, ew1_ref[...], preferred_element_type=jnp.float32)
            + eb1_ref[...], 0.0)
        ne2 = jnp.dot(h1, ew2_ref[...],
                      preferred_element_type=jnp.float32) + eb2_ref[...]
        oh1 = (lax.broadcasted_iota(jnp.int32, (NS, N2P), 0)
               == s2s_ref[...]).astype(jnp.float32)
        sub = jnp.dot(oh1, ne2, preferred_element_type=jnp.float32)
        h2 = jnp.maximum(
            jnp.dot(sub, nw1_ref[...], preferred_element_type=jnp.float32)
            + nb1_ref[...], 0.0)
        sub2 = jnp.dot(h2, nw2_ref[...],
                       preferred_element_type=jnp.float32) + nb2_ref[...]
        xf = jnp.maximum(
            jnp.dot(x_ref[...], pxw_ref[...],
                    preferred_element_type=jnp.float32) + pxb_ref[...], 0.0)
        oh2 = (lax.broadcasted_iota(jnp.int32, (G, NS), 0)
               == s2g_ref[...]).astype(jnp.float32)
        o_ref[...] = jnp.dot(oh2, sub2 * xf,
                             preferred_element_type=jnp.float32)

    return pl.pallas_call(
        body,
        out_shape=jax.ShapeDtypeStruct((G, H), jnp.float32),
    )(nsp, s2s_pad, s2g, x, pxW, pxb, ew1, eb1, ew2, eb2, nw1, nb1, nw2, nb2)


def _tc_hier3_post(nsp0, nsp1, nsp2, s2s_pad, s2g, x, pxW, pxb, wsets,
                   pw1, pb1, w2p, b2p):
    def body(*refs):
        (p0, p1, p2, s2s_ref, s2g_ref, x_ref, pxw_ref, pxb_ref) = refs[:8]
        wrefs = refs[8:32]
        pw1_ref, pb1_ref, w2_ref, b2_ref, o_ref = refs[32:]
        oh1 = (lax.broadcasted_iota(jnp.int32, (NS, N2P), 0)
               == s2s_ref[...]).astype(jnp.float32)
        oh2 = (lax.broadcasted_iota(jnp.int32, (G, NS), 0)
               == s2g_ref[...]).astype(jnp.float32)
        xf = jnp.maximum(
            jnp.dot(x_ref[...], pxw_ref[...],
                    preferred_element_type=jnp.float32) + pxb_ref[...], 0.0)
        e = jnp.zeros((G, H), jnp.float32)
        for li, p_ref in enumerate((p0, p1, p2)):
            (ew1_ref, eb1_ref, ew2_ref, eb2_ref,
             nw1_ref, nb1_ref, nw2_ref, nb2_ref) = wrefs[8 * li:8 * li + 8]
            ne = p_ref[0] + p_ref[1]
            h1 = jnp.maximum(
                jnp.dot(ne, ew1_ref[...], preferred_element_type=jnp.float32)
                + eb1_ref[...], 0.0)
            ne2 = jnp.dot(h1, ew2_ref[...],
                          preferred_element_type=jnp.float32) + eb2_ref[...]
            sub = jnp.dot(oh1, ne2, preferred_element_type=jnp.float32)
            h2 = jnp.maximum(
                jnp.dot(sub, nw1_ref[...], preferred_element_type=jnp.float32)
                + nb1_ref[...], 0.0)
            sub2 = jnp.dot(h2, nw2_ref[...],
                           preferred_element_type=jnp.float32) + nb2_ref[...]
            e = e + jnp.dot(oh2, sub2 * xf,
                            preferred_element_type=jnp.float32)
        hh = jnp.maximum(
            jnp.dot(e, pw1_ref[...], preferred_element_type=jnp.float32)
            + pb1_ref[...], 0.0)
        logits = jnp.dot(hh, w2_ref[...],
                         preferred_element_type=jnp.float32) + b2_ref[...]
        mx = jnp.max(logits, axis=1, keepdims=True)
        lse = jnp.log(jnp.sum(jnp.exp(logits - mx), axis=1,
                              keepdims=True)) + mx
        o_ref[...] = logits - lse

    args = ([nsp0, nsp1, nsp2, s2s_pad, s2g, x, pxW, pxb]
            + list(wsets) + [pw1, pb1, w2p, b2p])
    return pl.pallas_call(
        body,
        out_shape=jax.ShapeDtypeStruct((G, H), jnp.float32),
    )(*args)


def kernel(z, x, edge_index, batch, node_to_subgraph2, subgraph2_to_subgraph,
           subgraph_to_graph,
           emb, pxW, pxb, ie_w1, ie_b1, ie_w2, ie_b2,
           in_w1, in_b1, in_w2, in_b2,
           conv0_w, conv0_wih, conv0_whh, conv0_bih, conv0_bhh,
           e0_w1, e0_b1, e0_w2, e0_b2, n0_w1, n0_b1, n0_w2, n0_b2,
           conv1_w, conv1_wih, conv1_whh, conv1_bih, conv1_bhh,
           e1_w1, e1_b1, e1_w2, e1_b2, n1_w1, n1_b1, n1_w2, n1_b2,
           post_w1, post_b1, post_w2, post_b2):
    i32 = jnp.int32
    z_pad = jnp.concatenate([z.astype(i32), jnp.zeros((NP - N,), i32)])
    n2s2_pad = jnp.concatenate([node_to_subgraph2.astype(i32),
                                jnp.full((NP - N,), N2P - 1, i32)])
    s2s_pad = jnp.concatenate([subgraph2_to_subgraph.astype(i32),
                               jnp.full((N2P - N2,), NS, i32)]).reshape(1, N2P)
    s2g = subgraph_to_graph.astype(i32).reshape(1, NS)
    epw = E // _NWORK
    pad = _EPW - epw
    src = jnp.concatenate(
        [edge_index[0].astype(i32).reshape(_NWORK, epw),
         jnp.arange(_NWORK * pad, dtype=i32).reshape(_NWORK, pad) % N],
        axis=1).reshape(_NWORK, _NCH, _EC)
    dst = jnp.concatenate(
        [edge_index[1].astype(i32).reshape(_NWORK, epw),
         jnp.broadcast_to(jnp.arange(pad, dtype=i32)[None] + N,
                          (_NWORK, pad))], axis=1).reshape(_NWORK, _NCH, _EC)
    zero_np = jnp.zeros((NP, H), jnp.float32)
    zero_n2 = jnp.zeros((N2P, H), jnp.float32)

    r = lambda b: b.reshape(1, -1)
    w2p = jnp.concatenate([post_w2, jnp.zeros((H, H - C), jnp.float32)], 1)
    b2p = jnp.concatenate([post_b2,
                           jnp.full((H - C,), -1e30, jnp.float32)]).reshape(1, H)

    zraw = _sc_emb_gather(emb, z_pad)
    zf0, m0 = _tc_relu_mm(zraw, conv0_w)

    aggp0 = _sc_edge_agg(m0, src, dst, zero_np)
    zf1, m1 = _tc_gru_mm(aggp0, zf0, conv0_wih.T, conv0_whh.T,
                         r(conv0_bih), r(conv0_bhh), conv1_w)
    aggp1 = _sc_edge_agg(m1, src, dst, zero_np)
    zf2 = _tc_gru(aggp1, zf1, conv1_wih.T, conv1_whh.T,
                  r(conv1_bih), r(conv1_bhh))

    nsp0 = _sc_segsum(zf0, n2s2_pad, zero_n2)
    nsp1 = _sc_segsum(zf1, n2s2_pad, zero_n2)
    nsp2 = _sc_segsum(zf2, n2s2_pad, zero_n2)
    wsets = [ie_w1, r(ie_b1), ie_w2, r(ie_b2),
             in_w1, r(in_b1), in_w2, r(in_b2),
             e0_w1, r(e0_b1), e0_w2, r(e0_b2),
             n0_w1, r(n0_b1), n0_w2, r(n0_b2),
             e1_w1, r(e1_b1), e1_w2, r(e1_b2),
             n1_w1, r(n1_b1), n1_w2, r(n1_b2)]
    out = _tc_hier3_post(nsp0, nsp1, nsp2, s2s_pad, s2g, x, pxW, r(pxb),
                         wsets, post_w1, r(post_b1), w2p, b2p)
    return out[:, :C]


# speedup vs baseline: 1.0091x; 1.0091x over previous
import functools

import jax
import jax.numpy as jnp
from jax import lax
from jax.experimental import pallas as pl
from jax.experimental.pallas import tpu as pltpu
from jax.experimental.pallas import tpu_sc as plsc

H = 128
N = 10000
NP = 10240
E = 320000
N2 = 2000
N2P = 2048
NS = 400
G = 16
C = 10

_NWORK = 32
_CHUNK = 80

_mesh = plsc.VectorSubcoreMesh(core_axis_name="c", subcore_axis_name="s")


def _make_sc_gather(n_idx):
    per_w = n_idx // _NWORK
    n_chunks = per_w // _CHUNK

    @functools.partial(
        pl.kernel,
        out_type=jax.ShapeDtypeStruct((n_idx, H), jnp.float32),
        mesh=_mesh,
        scratch_types=[
            pltpu.VMEM((_CHUNK,), jnp.int32),
            pltpu.VMEM((_CHUNK, H), jnp.float32),
            pltpu.SemaphoreType.DMA,
        ],
    )
    def k(table_hbm, idx_hbm, out_hbm, idx_v, rows_v, sem):
        cid = lax.axis_index("c")
        sid = lax.axis_index("s")
        base = (sid * 2 + cid) * per_w

        def body(i, carry):
            off = base + i * _CHUNK
            pltpu.sync_copy(idx_hbm.at[pl.ds(off, _CHUNK)], idx_v)
            pltpu.async_copy(table_hbm.at[idx_v], rows_v, sem).wait()
            pltpu.sync_copy(rows_v, out_hbm.at[pl.ds(off, _CHUNK), :])
            return carry

        lax.fori_loop(0, n_chunks, body, 0)

    return k


def _make_sc_gather_pipe(n_idx):
    per_w = n_idx // _NWORK

    @functools.partial(
        pl.kernel,
        out_type=jax.ShapeDtypeStruct((n_idx, H), jnp.float32),
        mesh=_mesh,
        scratch_types=[
            pltpu.VMEM((_CHUNK,), jnp.int32),
            pltpu.VMEM((_CHUNK,), jnp.int32),
            pltpu.VMEM((_CHUNK, H), jnp.float32),
            pltpu.VMEM((_CHUNK, H), jnp.float32),
            pltpu.SemaphoreType.DMA,
            pltpu.SemaphoreType.DMA,
            pltpu.SemaphoreType.DMA,
            pltpu.SemaphoreType.DMA,
            pltpu.SemaphoreType.DMA,
            pltpu.SemaphoreType.DMA,
        ],
    )
    def k(table_hbm, idx_hbm, out_hbm, ix0, ix1, rw0, rw1,
          i0, i1, g0, g1, o0, o1):
        cid = lax.axis_index("c")
        sid = lax.axis_index("s")
        base = (sid * 2 + cid) * per_w
        ix = [ix0, ix1]
        rw = [rw0, rw1]
        isem = [i0, i1]
        gsem = [g0, g1]
        osem = [o0, o1]

        def off(j):
            return base + j * _CHUNK

        def fi(j):
            pltpu.async_copy(idx_hbm.at[pl.ds(off(j), _CHUNK)], ix[j % 2],
                             isem[j % 2])

        def wi(j):
            pltpu.make_async_copy(idx_hbm.at[pl.ds(off(j), _CHUNK)],
                                  ix[j % 2], isem[j % 2]).wait()

        def fg(j):
            pltpu.async_copy(table_hbm.at[ix[j % 2]], rw[j % 2], gsem[j % 2])

        def wg(j):
            pltpu.make_async_copy(table_hbm.at[ix[j % 2]], rw[j % 2],
                                  gsem[j % 2]).wait()

        def fo(j):
            pltpu.async_copy(rw[j % 2], out_hbm.at[pl.ds(off(j), _CHUNK), :],
                             osem[j % 2])

        def wo(j):
            pltpu.make_async_copy(rw[j % 2],
                                  out_hbm.at[pl.ds(off(j), _CHUNK), :],
                                  osem[j % 2]).wait()

        fi(0); fi(1)
        wi(0); fg(0)
        wi(1); fg(1)
        wg(0); fo(0); fi(2)
        wg(1); fo(1); fi(3)
        wo(0); wi(2); fg(2)
        wo(1); wi(3); fg(3)
        wg(2); fo(2)
        wg(3); fo(3)
        wo(2); wo(3)

    return k


_EC = 80
_EPW = 10240
EP = _NWORK * _EPW
_NCH = _EPW // _EC
_HCH = _NCH // 2
_HPAIR = _HCH // 2


def _make_sc_edge_agg():
    rpt = NP // 16

    @functools.partial(
        pl.kernel,
        out_type=jax.ShapeDtypeStruct((2, NP, H), jnp.float32),
        mesh=_mesh,
        scratch_types=(
            [pltpu.VMEM((_EC,), jnp.int32) for _ in range(16)]
            + [pltpu.VMEM((_EC, H), jnp.float32) for _ in range(4)]
            + [pltpu.VMEM_SHARED((NP, H), jnp.float32)]
            + [pltpu.SemaphoreType.DMA for _ in range(16)]
        ),
    )
    def k(m_hbm, src_hbm, dst_hbm, zero_hbm, out_hbm,
          si0, si1, si2, si3, si4, si5, si6, si7,
          di0, di1, di2, di3, di4, di5, di6, di7,
          rw0, rw1, rw2, rw3, acc,
          is0, is1, is2, is3, is4, is5, is6, is7,
          gg0, gg1, gg2, gg3, ss0, ss1, ss2, ss3):
        cid = lax.axis_index("c")
        sid = lax.axis_index("s")
        wid = sid * 2 + cid
        r0 = sid * rpt
        srcb = [si0, si1, si2, si3, si4, si5, si6, si7]
        dstb = [di0, di1, di2, di3, di4, di5, di6, di7]
        rows = [rw0, rw1, rw2, rw3]
        isem = [is0, is1, is2, is3, is4, is5, is6, is7]
        gsem = [gg0, gg1, gg2, gg3]
        ssem = [ss0, ss1, ss2, ss3]

        def fi(c, p):
            pltpu.async_copy(src_hbm.at[wid, c], srcb[p], isem[p])
            pltpu.async_copy(dst_hbm.at[wid, c], dstb[p], isem[p])

        def wi(c, p):
            pltpu.make_async_copy(src_hbm.at[wid, c], srcb[p],
                                  isem[p]).wait()
            pltpu.make_async_copy(dst_hbm.at[wid, c], dstb[p],
                                  isem[p]).wait()

        def fg(b, p):
            pltpu.async_copy(m_hbm.at[srcb[p]], rows[b], gsem[b])

        def wg(b, p):
            pltpu.make_async_copy(m_hbm.at[srcb[p]], rows[b], gsem[b]).wait()

        def fs(b, p):
            pltpu.async_copy(rows[b], acc.at[dstb[p]], ssem[b], add=True)

        def ws(b, p):
            pltpu.make_async_copy(rows[b], acc.at[dstb[p]], ssem[b]).wait()

        pltpu.sync_copy(zero_hbm.at[pl.ds(r0, rpt), :],
                        acc.at[pl.ds(r0, rpt), :])
        for p in range(6):
            fi(p, p)
        wi(0, 0)
        fg(0, 0)
        wi(1, 1)
        fg(1, 1)
        plsc.subcore_barrier()

        wg(0, 0); fs(0, 0); fi(6, 6); wi(2, 2); fg(2, 2)
        wg(1, 1); fs(1, 1); fi(7, 7); wi(3, 3); fg(3, 3)

        def body(i, carry):
            cbase = 2 + 8 * i
            for j in range(8):
                c = cbase + j
                b = (2 + j) % 4
                q = (2 + j) % 8
                bn = j % 4
                qn = (4 + j) % 8
                wg(b, q)
                fs(b, q)
                ws(bn, j)
                fi(c + 6, j)
                wi(c + 2, qn)
                fg(bn, qn)
            return carry

        lax.fori_loop(0, (_NCH - 8) // 8, body, 0)

        wg(2, 2); fs(2, 2); ws(0, 0); wi(124, 4); fg(0, 4)
        wg(3, 3); fs(3, 3); ws(1, 1); wi(125, 5); fg(1, 5)
        wg(0, 4); fs(0, 4); ws(2, 2); wi(126, 6); fg(2, 6)
        wg(1, 5); fs(1, 5); ws(3, 3); wi(127, 7); fg(3, 7)
        wg(2, 6); fs(2, 6); ws(0, 4)
        wg(3, 7); fs(3, 7); ws(1, 5)
        ws(2, 6)
        ws(3, 7)

        plsc.subcore_barrier()
        pltpu.sync_copy(acc.at[pl.ds(r0, rpt), :],
                        out_hbm.at[cid, pl.ds(r0, rpt), :])

    return k


def _make_sc_segsum(n_rows, n_seg):
    per_w = n_rows // _NWORK
    n_chunks = per_w // _CHUNK
    rpt = n_seg // 16

    @functools.partial(
        pl.kernel,
        out_type=jax.ShapeDtypeStruct((2, n_seg, H), jnp.float32),
        mesh=_mesh,
        scratch_types=[
            pltpu.VMEM((_CHUNK,), jnp.int32),
            pltpu.VMEM((_CHUNK,), jnp.int32),
            pltpu.VMEM((_CHUNK, H), jnp.float32),
            pltpu.VMEM((_CHUNK, H), jnp.float32),
            pltpu.VMEM_SHARED((n_seg, H), jnp.float32),
            pltpu.SemaphoreType.DMA,
            pltpu.SemaphoreType.DMA,
            pltpu.SemaphoreType.DMA,
            pltpu.SemaphoreType.DMA,
        ],
    )
    def k(x_hbm, idx_hbm, zero_hbm, out_hbm, ix0, ix1, rw0, rw1, acc,
          i0, i1, s0, s1):
        cid = lax.axis_index("c")
        sid = lax.axis_index("s")
        r0 = sid * rpt
        base = (sid * 2 + cid) * per_w
        ix = [ix0, ix1]
        rw = [rw0, rw1]
        isem = [i0, i1]
        ssem = [s0, s1]

        def off(j):
            return base + j * _CHUNK

        def fl(j):
            pltpu.async_copy(idx_hbm.at[pl.ds(off(j), _CHUNK)], ix[j % 2],
                             isem[j % 2])
            pltpu.async_copy(x_hbm.at[pl.ds(off(j), _CHUNK), :], rw[j % 2],
                             isem[j % 2])

        def wl(j):
            pltpu.make_async_copy(idx_hbm.at[pl.ds(off(j), _CHUNK)],
                                  ix[j % 2], isem[j % 2]).wait()
            pltpu.make_async_copy(x_hbm.at[pl.ds(off(j), _CHUNK), :],
                                  rw[j % 2], isem[j % 2]).wait()

        def fs(j):
            pltpu.async_copy(rw[j % 2], acc.at[ix[j % 2]], ssem[j % 2],
                             add=True)

        def ws(j):
            pltpu.make_async_copy(rw[j % 2], acc.at[ix[j % 2]],
                                  ssem[j % 2]).wait()

        fl(0); fl(1)
        pltpu.sync_copy(zero_hbm.at[pl.ds(r0, rpt), :],
                        acc.at[pl.ds(r0, rpt), :])
        plsc.subcore_barrier()
        wl(0); fs(0)
        wl(1); fs(1)
        ws(0); fl(2)
        ws(1); fl(3)
        wl(2); fs(2)
        wl(3); fs(3)
        ws(2); ws(3)
        plsc.subcore_barrier()
        pltpu.sync_copy(acc.at[pl.ds(r0, rpt), :],
                        out_hbm.at[cid, pl.ds(r0, rpt), :])

    return k


_sc_emb_gather = _make_sc_gather_pipe(NP)
_sc_edge_agg = _make_sc_edge_agg()
_sc_segsum = _make_sc_segsum(NP, N2P)


_BN = 1024


def _tc_relu_mm(zraw, w):
    def body(z_ref, w_ref, zf_ref, m_ref):
        zf = jnp.maximum(z_ref[...], 0.0)
        zf_ref[...] = zf
        m_ref[...] = jnp.dot(zf, w_ref[...], preferred_element_type=jnp.float32)

    return pl.pallas_call(
        body,
        grid=(NP // _BN,),
        in_specs=[pl.BlockSpec((_BN, H), lambda i: (i, 0)),
                  pl.BlockSpec((H, H), lambda i: (0, 0))],
        out_specs=[pl.BlockSpec((_BN, H), lambda i: (i, 0)),
                   pl.BlockSpec((_BN, H), lambda i: (i, 0))],
        out_shape=[jax.ShapeDtypeStruct((NP, H), jnp.float32),
                   jax.ShapeDtypeStruct((NP, H), jnp.float32)],
    )(zraw, w)


def _tc_mm(zf, w):
    def body(z_ref, w_ref, m_ref):
        m_ref[...] = jnp.dot(z_ref[...], w_ref[...],
                             preferred_element_type=jnp.float32)

    return pl.pallas_call(
        body,
        grid=(NP // _BN,),
        in_specs=[pl.BlockSpec((_BN, H), lambda i: (i, 0)),
                  pl.BlockSpec((H, H), lambda i: (0, 0))],
        out_specs=pl.BlockSpec((_BN, H), lambda i: (i, 0)),
        out_shape=jax.ShapeDtypeStruct((NP, H), jnp.float32),
    )(zf, w)


def _tc_gru(aggp, h, wihT, whhT, bih, bhh):
    def body(p_ref, h_ref, wih_ref, whh_ref, bih_ref, bhh_ref, o_ref):
        agg = p_ref[0] + p_ref[1]
        hh = h_ref[...]
        gi = jnp.dot(agg, wih_ref[...],
                     preferred_element_type=jnp.float32) + bih_ref[...]
        gh = jnp.dot(hh, whh_ref[...],
                     preferred_element_type=jnp.float32) + bhh_ref[...]
        r = jax.nn.sigmoid(gi[:, :H] + gh[:, :H])
        u = jax.nn.sigmoid(gi[:, H:2 * H] + gh[:, H:2 * H])
        nn_ = jnp.tanh(gi[:, 2 * H:] + r * gh[:, 2 * H:])
        o_ref[...] = jnp.maximum((1.0 - u) * nn_ + u * hh, 0.0)

    return pl.pallas_call(
        body,
        grid=(NP // _BN,),
        in_specs=[pl.BlockSpec((2, _BN, H), lambda i: (0, i, 0)),
                  pl.BlockSpec((_BN, H), lambda i: (i, 0)),
                  pl.BlockSpec((H, 3 * H), lambda i: (0, 0)),
                  pl.BlockSpec((H, 3 * H), lambda i: (0, 0)),
                  pl.BlockSpec((1, 3 * H), lambda i: (0, 0)),
                  pl.BlockSpec((1, 3 * H), lambda i: (0, 0))],
        out_specs=pl.BlockSpec((_BN, H), lambda i: (i, 0)),
        out_shape=jax.ShapeDtypeStruct((NP, H), jnp.float32),
    )(aggp, h, wihT, whhT, bih, bhh)


def _tc_gru_mm(aggp, h, wihT, whhT, bih, bhh, wnext):
    def body(p_ref, h_ref, wih_ref, whh_ref, bih_ref, bhh_ref, wn_ref,
             o_ref, m_ref):
        agg = p_ref[0] + p_ref[1]
        hh = h_ref[...]
        gi = jnp.dot(agg, wih_ref[...],
                     preferred_element_type=jnp.float32) + bih_ref[...]
        gh = jnp.dot(hh, whh_ref[...],
                     preferred_element_type=jnp.float32) + bhh_ref[...]
        r = jax.nn.sigmoid(gi[:, :H] + gh[:, :H])
        u = jax.nn.sigmoid(gi[:, H:2 * H] + gh[:, H:2 * H])
        nn_ = jnp.tanh(gi[:, 2 * H:] + r * gh[:, 2 * H:])
        zf = jnp.maximum((1.0 - u) * nn_ + u * hh, 0.0)
        o_ref[...] = zf
        m_ref[...] = jnp.dot(zf, wn_ref[...],
                             preferred_element_type=jnp.float32)

    return pl.pallas_call(
        body,
        grid=(NP // _BN,),
        in_specs=[pl.BlockSpec((2, _BN, H), lambda i: (0, i, 0)),
                  pl.BlockSpec((_BN, H), lambda i: (i, 0)),
                  pl.BlockSpec((H, 3 * H), lambda i: (0, 0)),
                  pl.BlockSpec((H, 3 * H), lambda i: (0, 0)),
                  pl.BlockSpec((1, 3 * H), lambda i: (0, 0)),
                  pl.BlockSpec((1, 3 * H), lambda i: (0, 0)),
                  pl.BlockSpec((H, H), lambda i: (0, 0))],
        out_specs=[pl.BlockSpec((_BN, H), lambda i: (i, 0)),
                   pl.BlockSpec((_BN, H), lambda i: (i, 0))],
        out_shape=[jax.ShapeDtypeStruct((NP, H), jnp.float32),
                   jax.ShapeDtypeStruct((NP, H), jnp.float32)],
    )(aggp, h, wihT, whhT, bih, bhh, wnext)


def _tc_hier(nsp, s2s_pad, s2g, x, pxW, pxb, ew1, eb1, ew2, eb2,
             nw1, nb1, nw2, nb2):
    def body(p_ref, s2s_ref, s2g_ref, x_ref, pxw_ref, pxb_ref,
             ew1_ref, eb1_ref, ew2_ref, eb2_ref,
             nw1_ref, nb1_ref, nw2_ref, nb2_ref, o_ref):
        ne = p_ref[0] + p_ref[1]
        h1 = jnp.maximum(
            jnp.dot(ne, ew1_ref[...], preferred_element_type=jnp.float32)
            + eb1_ref[...], 0.0)
        ne2 = jnp.dot(h1, ew2_ref[...],
                      preferred_element_type=jnp.float32) + eb2_ref[...]
        oh1 = (lax.broadcasted_iota(jnp.int32, (NS, N2P), 0)
               == s2s_ref[...]).astype(jnp.float32)
        sub = jnp.dot(oh1, ne2, preferred_element_type=jnp.float32)
        h2 = jnp.maximum(
            jnp.dot(sub, nw1_ref[...], preferred_element_type=jnp.float32)
            + nb1_ref[...], 0.0)
        sub2 = jnp.dot(h2, nw2_ref[...],
                       preferred_element_type=jnp.float32) + nb2_ref[...]
        xf = jnp.maximum(
            jnp.dot(x_ref[...], pxw_ref[...],
                    preferred_element_type=jnp.float32) + pxb_ref[...], 0.0)
        oh2 = (lax.broadcasted_iota(jnp.int32, (G, NS), 0)
               == s2g_ref[...]).astype(jnp.float32)
        o_ref[...] = jnp.dot(oh2, sub2 * xf,
                             preferred_element_type=jnp.float32)

    return pl.pallas_call(
        body,
        out_shape=jax.ShapeDtypeStruct((G, H), jnp.float32),
    )(nsp, s2s_pad, s2g, x, pxW, pxb, ew1, eb1, ew2, eb2, nw1, nb1, nw2, nb2)


def _tc_post(ges, w1, b1, w2p, b2p):
    def body(g_ref, w1_ref, b1_ref, w2_ref, b2_ref, o_ref):
        e = g_ref[0] + g_ref[1] + g_ref[2]
        hh = jnp.maximum(
            jnp.dot(e, w1_ref[...], preferred_element_type=jnp.float32)
            + b1_ref[...], 0.0)
        logits = jnp.dot(hh, w2_ref[...],
                         preferred_element_type=jnp.float32) + b2_ref[...]
        mx = jnp.max(logits, axis=1, keepdims=True)
        lse = jnp.log(jnp.sum(jnp.exp(logits - mx), axis=1,
                              keepdims=True)) + mx
        o_ref[...] = logits - lse

    return pl.pallas_call(
        body,
        out_shape=jax.ShapeDtypeStruct((G, H), jnp.float32),
    )(ges, w1, b1, w2p, b2p)


def kernel(z, x, edge_index, batch, node_to_subgraph2, subgraph2_to_subgraph,
           subgraph_to_graph,
           emb, pxW, pxb, ie_w1, ie_b1, ie_w2, ie_b2,
           in_w1, in_b1, in_w2, in_b2,
           conv0_w, conv0_wih, conv0_whh, conv0_bih, conv0_bhh,
           e0_w1, e0_b1, e0_w2, e0_b2, n0_w1, n0_b1, n0_w2, n0_b2,
           conv1_w, conv1_wih, conv1_whh, conv1_bih, conv1_bhh,
           e1_w1, e1_b1, e1_w2, e1_b2, n1_w1, n1_b1, n1_w2, n1_b2,
           post_w1, post_b1, post_w2, post_b2):
    i32 = jnp.int32
    z_pad = jnp.concatenate([z.astype(i32), jnp.zeros((NP - N,), i32)])
    n2s2_pad = jnp.concatenate([node_to_subgraph2.astype(i32),
                                jnp.full((NP - N,), N2P - 1, i32)])
    s2s_pad = jnp.concatenate([subgraph2_to_subgraph.astype(i32),
                               jnp.full((N2P - N2,), NS, i32)]).reshape(1, N2P)
    s2g = subgraph_to_graph.astype(i32).reshape(1, NS)
    epw = E // _NWORK
    pad = _EPW - epw
    src = jnp.concatenate(
        [edge_index[0].astype(i32).reshape(_NWORK, epw),
         jnp.arange(_NWORK * pad, dtype=i32).reshape(_NWORK, pad) % N],
        axis=1).reshape(_NWORK, _NCH, _EC)
    dst = jnp.concatenate(
        [edge_index[1].astype(i32).reshape(_NWORK, epw),
         jnp.broadcast_to(jnp.arange(pad, dtype=i32)[None] + N,
                          (_NWORK, pad))], axis=1).reshape(_NWORK, _NCH, _EC)
    zero_np = jnp.zeros((NP, H), jnp.float32)
    zero_n2 = jnp.zeros((N2P, H), jnp.float32)

    r = lambda b: b.reshape(1, -1)
    w2p = jnp.concatenate([post_w2, jnp.zeros((H, H - C), jnp.float32)], 1)
    b2p = jnp.concatenate([post_b2,
                           jnp.full((H - C,), -1e30, jnp.float32)]).reshape(1, H)

    zraw = _sc_emb_gather(emb, z_pad)
    zf0, m0 = _tc_relu_mm(zraw, conv0_w)

    aggp0 = _sc_edge_agg(m0, src, dst, zero_np)
    zf1, m1 = _tc_gru_mm(aggp0, zf0, conv0_wih.T, conv0_whh.T,
                         r(conv0_bih), r(conv0_bhh), conv1_w)
    aggp1 = _sc_edge_agg(m1, src, dst, zero_np)
    zf2 = _tc_gru(aggp1, zf1, conv1_wih.T, conv1_whh.T,
                  r(conv1_bih), r(conv1_bhh))

    ges = []
    for zf, ws in ((zf0, (ie_w1, ie_b1, ie_w2, ie_b2,
                          in_w1, in_b1, in_w2, in_b2)),
                   (zf1, (e0_w1, e0_b1, e0_w2, e0_b2,
                          n0_w1, n0_b1, n0_w2, n0_b2)),
                   (zf2, (e1_w1, e1_b1, e1_w2, e1_b2,
                          n1_w1, n1_b1, n1_w2, n1_b2))):
        w1, b1, w2, b2, v1, c1, v2, c2 = ws
        nsp = _sc_segsum(zf, n2s2_pad, zero_n2)
        ges.append(_tc_hier(nsp, s2s_pad, s2g, x, pxW, r(pxb),
                            w1, r(b1), w2, r(b2), v1, r(c1), v2, r(c2)))

    out = _tc_post(jnp.stack(ges), post_w1, r(post_b1), w2p, b2p)
    return out[:, :C]
